# baseline (speedup 1.0000x reference)
import functools

import jax
import jax.numpy as jnp
from jax import lax
from jax.experimental import pallas as pl
from jax.experimental.pallas import tpu as pltpu
from jax.experimental.pallas import tpu_sc as plsc

N = 4096
F = 512
L = 16
H1 = 256
H2 = 128
E = 131072
TEMP = 0.5

BM1 = 512


def _h1_body(fn_ref, w1_ref, wy_ref, h1_ref, h1b_ref, prob_ref):
    h = jnp.tanh(
        jax.lax.dot_general(
            fn_ref[...], w1_ref[...], (((1,), (0,)), ((), ())),
            preferred_element_type=jnp.float32,
        )
    )
    h1_ref[...] = h
    h1b_ref[...] = h
    logits = jax.lax.dot_general(
        h, wy_ref[...], (((1,), (0,)), ((), ())),
        preferred_element_type=jnp.float32,
    )
    m = jnp.max(logits, axis=1, keepdims=True)
    e = jnp.exp(logits - m)
    prob_ref[...] = e / jnp.sum(e, axis=1, keepdims=True)


def _hidden1(Fn, W1, Wy):
    return pl.pallas_call(
        _h1_body,
        grid=(N // BM1,),
        in_specs=[
            pl.BlockSpec((BM1, N + F), lambda i: (i, 0)),
            pl.BlockSpec((N + F, H1), lambda i: (0, 0)),
            pl.BlockSpec((H1, L), lambda i: (0, 0)),
        ],
        out_specs=[
            pl.BlockSpec((BM1, H1), lambda i: (i, 0)),
            pl.BlockSpec((BM1, H1), lambda i: (i, 0)),
            pl.BlockSpec((BM1, L), lambda i: (i, 0)),
        ],
        out_shape=[
            jax.ShapeDtypeStruct((N, H1), jnp.float32),
            jax.ShapeDtypeStruct((N, H1), jnp.float32),
            jax.ShapeDtypeStruct((N, L), jnp.float32),
        ],
    )(Fn, W1, Wy)


NC, NS = 2, 16
CHUNK = 256
NCHUNKS = E // CHUNK
SC_SUP = 8
NSUP = E // (NC * SC_SUP * CHUNK)
NBUF = 4


def _bcast16(x, k):
    return jax.lax.gather(
        x, jnp.full((16, 1), k, jnp.int32),
        jax.lax.GatherDimensionNumbers(
            offset_dims=(), collapsed_slice_dims=(0,), start_index_map=(0,)),
        slice_sizes=(1,),
        mode=jax.lax.GatherScatterMode.PROMISE_IN_BOUNDS)


def _sc_scatter_body(h1_hbm, h1b_hbm, rows_hbm, cols_hbm, out_hbm,
                     ridx0, ridx1, cidx0, cidx1,
                     gb0, gb1, gb2, gb3, acc,
                     ssem0, ssem1, gsem0, gsem1, gsem2, gsem3):
    c = lax.axis_index("c")
    s = lax.axis_index("s")
    c0 = s * 16
    zeros16 = jnp.zeros((16,), jnp.float32)
    iota16 = jax.lax.iota(jnp.int32, 16)

    def zbody(r, _):
        acc[r] = zeros16
        return 0

    lax.fori_loop(0, N, zbody, 0)

    ridx = (ridx0, ridx1)
    cidx = (cidx0, cidx1)
    ssem = (ssem0, ssem1)
    gb = (gb0, gb1, gb2, gb3)
    gsem = (gsem0, gsem1, gsem2, gsem3)
    base = c * NSUP

    def stage(m, p):
        pltpu.async_copy(rows_hbm.at[pl.ds(m * SC_SUP, SC_SUP)], ridx[p],
                         ssem[p])
        pltpu.async_copy(cols_hbm.at[pl.ds(m * SC_SUP, SC_SUP)], cidx[p],
                         ssem[p])

    def stage_wait(p):
        pltpu.make_async_copy(rows_hbm.at[pl.ds(0, SC_SUP)], ridx[p],
                              ssem[p]).wait()
        pltpu.make_async_copy(cols_hbm.at[pl.ds(0, SC_SUP)], cidx[p],
                              ssem[p]).wait()

    def gissue(p, j, b):
        @pl.when(c == 0)
        def _():
            pltpu.async_copy(h1_hbm.at[cidx[p].at[j]], gb[b], gsem[b])

        @pl.when(c == 1)
        def _():
            pltpu.async_copy(h1b_hbm.at[cidx[p].at[j]], gb[b], gsem[b])

    def gwait(p, j, b):
        pltpu.make_async_copy(h1_hbm.at[cidx[p].at[j]], gb[b],
                              gsem[b]).wait()

    def accumulate(p, j, b):
        def acc_body(g, _):
            rv = ridx[p][j, pl.ds(g * 16, 16)]
            vals = [gb[b][g * 16 + k] for k in range(16)]
            idxs = [_bcast16(rv, k) for k in range(16)]
            for k in range(16):
                plsc.addupdate_scatter(acc, [idxs[k], iota16], vals[k])
            return 0

        lax.fori_loop(0, CHUNK // 16, acc_body, 0)

    stage(base, 0)
    stage(base + 1, 1)

    def sup_body(i, _):
        for p in range(2):
            m = 2 * i + p
            stage_wait(p)
            def xform(g, _):
                for q in range(CHUNK // 16):
                    cidx[p][g, pl.ds(q * 16, 16)] = (
                        cidx[p][g, pl.ds(q * 16, 16)] * 16 + s)
                return 0

            lax.fori_loop(0, SC_SUP, xform, 0)
            for b in range(NBUF):
                gissue(p, b, b)

            def chunk_body(t, _):
                for b in range(NBUF):
                    j = t * NBUF + b
                    gwait(p, j, b)
                    accumulate(p, j, b)
                    gissue(p, j + NBUF, b)
                return 0

            lax.fori_loop(0, (SC_SUP - NBUF) // NBUF, chunk_body, 0)
            for b in range(NBUF):
                j = SC_SUP - NBUF + b
                gwait(p, j, b)
                accumulate(p, j, b)
            @pl.when(m + 2 < NSUP)
            def _():
                stage(base + m + 2, p)
        return 0

    lax.fori_loop(0, NSUP // 2, sup_body, 0)
    pltpu.sync_copy(acc, out_hbm.at[c, :, pl.ds(c0, 16)])


def _node_fea_partials(hidden1, hidden1b, rows, cols):
    mesh = plsc.VectorSubcoreMesh(core_axis_name="c", subcore_axis_name="s")
    return pl.kernel(
        _sc_scatter_body,
        out_type=jax.ShapeDtypeStruct((NC, N, H1), jnp.float32),
        mesh=mesh,
        compiler_params=pltpu.CompilerParams(use_tc_tiling_on_sc=False,
                                             needs_layout_passes=False),
        scratch_types=[
            pltpu.VMEM((SC_SUP, CHUNK), jnp.int32),
            pltpu.VMEM((SC_SUP, CHUNK), jnp.int32),
            pltpu.VMEM((SC_SUP, CHUNK), jnp.int32),
            pltpu.VMEM((SC_SUP, CHUNK), jnp.int32),
            pltpu.VMEM((CHUNK, 16), jnp.float32),
            pltpu.VMEM((CHUNK, 16), jnp.float32),
            pltpu.VMEM((CHUNK, 16), jnp.float32),
            pltpu.VMEM((CHUNK, 16), jnp.float32),
            pltpu.VMEM((N, 16), jnp.float32),
            pltpu.SemaphoreType.DMA,
            pltpu.SemaphoreType.DMA,
            pltpu.SemaphoreType.DMA,
            pltpu.SemaphoreType.DMA,
            pltpu.SemaphoreType.DMA,
            pltpu.SemaphoreType.DMA,
        ],
    )(hidden1.reshape(N * 16, 16), hidden1b.reshape(N * 16, 16),
      rows.reshape(NCHUNKS, CHUNK), cols.reshape(NCHUNKS, CHUNK))


def _za_body(fat_ref, w2_ref, wam_ref, was_ref, eps_ref, za_ref):
    h2 = jnp.tanh(
        jax.lax.dot_general(
            fat_ref[...], w2_ref[...], (((1,), (0,)), ((), ())),
            preferred_element_type=jnp.float32,
        )
    )
    mean = jax.lax.dot_general(
        h2, wam_ref[...], (((1,), (0,)), ((), ())),
        preferred_element_type=jnp.float32,
    )
    logstd = jax.lax.dot_general(
        h2, was_ref[...], (((1,), (0,)), ((), ())),
        preferred_element_type=jnp.float32,
    )
    za_ref[...] = mean + eps_ref[...] * jnp.exp(logstd)


def _z_a(FaT, W2, Wam, Was, eps_a):
    return pl.pallas_call(
        _za_body,
        out_shape=jax.ShapeDtypeStruct((F, H2 + L), jnp.float32),
    )(FaT, W2, Wam, Was, eps_a)


BM3 = 512


def _dec_body(p0_ref, p1_ref, prob_ref, lp_ref, yt_ref, gum_ref, eps_ref,
              wum_ref, wus_ref, za_ref, out_ref):
    node_fea = p0_ref[...] + p1_ref[...]
    prob = prob_ref[...]
    lp = lp_ref[...]
    yt = yt_ref[...]
    yz = lp * yt + (1.0 - lp) * prob

    wum = wum_ref[...]
    wus = wus_ref[...]
    z_u_mean = (
        jax.lax.dot_general(node_fea, wum[:H1], (((1,), (0,)), ((), ())),
                            preferred_element_type=jnp.float32)
        + jax.lax.dot_general(yz, wum[H1:], (((1,), (0,)), ((), ())),
                              preferred_element_type=jnp.float32)
    )
    z_u_log = (
        jax.lax.dot_general(node_fea, wus[:H1], (((1,), (0,)), ((), ())),
                            preferred_element_type=jnp.float32)
        + jax.lax.dot_general(yz, wus[H1:], (((1,), (0,)), ((), ())),
                              preferred_element_type=jnp.float32)
    )
    z_u = z_u_mean + eps_ref[...] * jnp.exp(z_u_log)

    g = -jnp.log(-jnp.log(gum_ref[...]))
    yp = jnp.exp((jnp.log(prob) + g) / TEMP)
    yp = yp / jnp.sum(yp, axis=1, keepdims=True)
    y_rec = lp * yt + (1.0 - lp) * yp

    za = za_ref[...]
    out_ref[...] = (
        jax.lax.dot_general(z_u, za[:, :H2], (((1,), (1,)), ((), ())),
                            preferred_element_type=jnp.float32)
        + jax.lax.dot_general(y_rec, za[:, H2:], (((1,), (1,)), ((), ())),
                              preferred_element_type=jnp.float32)
    )


def _decode(p0, p1, prob, lp, y_train, gumbel_u, eps_u, Wum, Wus, z_a):
    return pl.pallas_call(
        _dec_body,
        grid=(N // BM3,),
        in_specs=[
            pl.BlockSpec((BM3, H1), lambda i: (i, 0)),
            pl.BlockSpec((BM3, H1), lambda i: (i, 0)),
            pl.BlockSpec((BM3, L), lambda i: (i, 0)),
            pl.BlockSpec((BM3, L), lambda i: (i, 0)),
            pl.BlockSpec((BM3, L), lambda i: (i, 0)),
            pl.BlockSpec((BM3, L), lambda i: (i, 0)),
            pl.BlockSpec((BM3, H2), lambda i: (i, 0)),
            pl.BlockSpec((H1 + L, H2), lambda i: (0, 0)),
            pl.BlockSpec((H1 + L, H2), lambda i: (0, 0)),
            pl.BlockSpec((F, H2 + L), lambda i: (0, 0)),
        ],
        out_specs=pl.BlockSpec((BM3, F), lambda i: (i, 0)),
        out_shape=jax.ShapeDtypeStruct((N, F), jnp.float32),
    )(p0, p1, prob, lp, y_train, gumbel_u, eps_u, Wum, Wus, z_a)


def kernel(Fn, FaT, edge_index, labels_pos, y_train, gumbel_u, eps_u, eps_a,
           W1, Wy, W2, Wum, Wus, Wam, Was):
    hidden1, hidden1b, prob = _hidden1(Fn, W1, Wy)
    rows = edge_index[0].astype(jnp.int32)
    cols = edge_index[1].astype(jnp.int32)
    partials = _node_fea_partials(hidden1, hidden1b, rows, cols)
    z_a = _z_a(FaT, W2, Wam, Was, eps_a)
    lp = labels_pos.astype(jnp.float32)
    recon = _decode(partials[0], partials[1], prob, lp, y_train, gumbel_u,
                    eps_u, Wum, Wus, z_a)
    return recon.reshape(-1)


# speedup vs baseline: 1.0525x; 1.0525x over previous
import functools

import jax
import jax.numpy as jnp
from jax import lax
from jax.experimental import pallas as pl
from jax.experimental.pallas import tpu as pltpu
from jax.experimental.pallas import tpu_sc as plsc

N = 4096
F = 512
L = 16
H1 = 256
H2 = 128
E = 131072
TEMP = 0.5

BM1 = 512


def _h1_body(fn_ref, w1_ref, wy_ref, h1_ref, prob_ref):
    h = jnp.tanh(
        jax.lax.dot_general(
            fn_ref[...], w1_ref[...], (((1,), (0,)), ((), ())),
            preferred_element_type=jnp.float32,
        )
    )
    h1_ref[...] = h
    logits = jax.lax.dot_general(
        h, wy_ref[...], (((1,), (0,)), ((), ())),
        preferred_element_type=jnp.float32,
    )
    m = jnp.max(logits, axis=1, keepdims=True)
    e = jnp.exp(logits - m)
    prob_ref[...] = e / jnp.sum(e, axis=1, keepdims=True)


def _hidden1(Fn, W1, Wy):
    return pl.pallas_call(
        _h1_body,
        grid=(N // BM1,),
        in_specs=[
            pl.BlockSpec((BM1, N + F), lambda i: (i, 0)),
            pl.BlockSpec((N + F, H1), lambda i: (0, 0)),
            pl.BlockSpec((H1, L), lambda i: (0, 0)),
        ],
        out_specs=[
            pl.BlockSpec((BM1, H1), lambda i: (i, 0)),
            pl.BlockSpec((BM1, L), lambda i: (i, 0)),
        ],
        out_shape=[
            jax.ShapeDtypeStruct((N, H1), jnp.float32),
            jax.ShapeDtypeStruct((N, L), jnp.float32),
        ],
    )(Fn, W1, Wy)


NC, NS = 2, 16
CHUNK = 256
NCHUNKS = E // CHUNK
SC_SUP = 8
NSUP = E // (NC * SC_SUP * CHUNK)
NBUF = 8


def _bcast16(x, k):
    return jax.lax.gather(
        x, jnp.full((16, 1), k, jnp.int32),
        jax.lax.GatherDimensionNumbers(
            offset_dims=(), collapsed_slice_dims=(0,), start_index_map=(0,)),
        slice_sizes=(1,),
        mode=jax.lax.GatherScatterMode.PROMISE_IN_BOUNDS)


def _sc_scatter_body(h1_hbm, rows_hbm, cols_hbm, out_hbm,
                     ridx0, ridx1, cidx0, cidx1,
                     gb0, gb1, gb2, gb3, gb4, gb5, gb6, gb7, acc,
                     ssem0, ssem1, gsem0, gsem1, gsem2, gsem3,
                     gsem4, gsem5, gsem6, gsem7):
    c = lax.axis_index("c")
    s = lax.axis_index("s")
    c0 = s * 16
    zeros16 = jnp.zeros((16,), jnp.float32)
    iota16 = jax.lax.iota(jnp.int32, 16)

    def zbody(r, _):
        acc[r] = zeros16
        return 0

    lax.fori_loop(0, N, zbody, 0)

    ridx = (ridx0, ridx1)
    cidx = (cidx0, cidx1)
    ssem = (ssem0, ssem1)
    gb = (gb0, gb1, gb2, gb3, gb4, gb5, gb6, gb7)
    gsem = (gsem0, gsem1, gsem2, gsem3, gsem4, gsem5, gsem6, gsem7)
    base = c * NSUP

    def stage(m, p):
        pltpu.async_copy(rows_hbm.at[pl.ds(m * SC_SUP, SC_SUP)], ridx[p],
                         ssem[p])
        pltpu.async_copy(cols_hbm.at[pl.ds(m * SC_SUP, SC_SUP)], cidx[p],
                         ssem[p])

    def stage_wait(p):
        pltpu.make_async_copy(rows_hbm.at[pl.ds(0, SC_SUP)], ridx[p],
                              ssem[p]).wait()
        pltpu.make_async_copy(cols_hbm.at[pl.ds(0, SC_SUP)], cidx[p],
                              ssem[p]).wait()

    def gissue(p, j, b):
        pltpu.async_copy(h1_hbm.at[cidx[p].at[j]], gb[b], gsem[b])

    def gwait(p, j, b):
        pltpu.make_async_copy(h1_hbm.at[cidx[p].at[j]], gb[b],
                              gsem[b]).wait()

    def accumulate(p, j, b):
        def acc_body(g, _):
            rv = ridx[p][j, pl.ds(g * 16, 16)]
            vals = [gb[b][g * 16 + k] for k in range(16)]
            idxs = [_bcast16(rv, k) for k in range(16)]
            for k in range(16):
                plsc.addupdate_scatter(acc, [idxs[k], iota16], vals[k])
            return 0

        lax.fori_loop(0, CHUNK // 16, acc_body, 0)

    stage(base, 0)
    stage(base + 1, 1)

    def sup_body(i, _):
        for p in range(2):
            m = 2 * i + p
            stage_wait(p)
            def xform(g, _):
                for q in range(CHUNK // 16):
                    cidx[p][g, pl.ds(q * 16, 16)] = (
                        cidx[p][g, pl.ds(q * 16, 16)] * 16 + s)
                return 0

            lax.fori_loop(0, SC_SUP, xform, 0)
            for b in range(NBUF):
                gissue(p, b, b)

            def chunk_body(t, _):
                for b in range(NBUF):
                    j = t * NBUF + b
                    gwait(p, j, b)
                    accumulate(p, j, b)
                    gissue(p, j + NBUF, b)
                return 0

            lax.fori_loop(0, (SC_SUP - NBUF) // NBUF, chunk_body, 0)
            for b in range(NBUF):
                j = SC_SUP - NBUF + b
                gwait(p, j, b)
                accumulate(p, j, b)
            @pl.when(m + 2 < NSUP)
            def _():
                stage(base + m + 2, p)
        return 0

    lax.fori_loop(0, NSUP // 2, sup_body, 0)
    pltpu.sync_copy(acc, out_hbm.at[c, :, pl.ds(c0, 16)])


def _node_fea_partials(hidden1, rows, cols):
    mesh = plsc.VectorSubcoreMesh(core_axis_name="c", subcore_axis_name="s")
    return pl.kernel(
        _sc_scatter_body,
        out_type=jax.ShapeDtypeStruct((NC, N, H1), jnp.float32),
        mesh=mesh,
        compiler_params=pltpu.CompilerParams(use_tc_tiling_on_sc=False,
                                             needs_layout_passes=False),
        scratch_types=[
            pltpu.VMEM((SC_SUP, CHUNK), jnp.int32),
            pltpu.VMEM((SC_SUP, CHUNK), jnp.int32),
            pltpu.VMEM((SC_SUP, CHUNK), jnp.int32),
            pltpu.VMEM((SC_SUP, CHUNK), jnp.int32),
            pltpu.VMEM((CHUNK, 16), jnp.float32),
            pltpu.VMEM((CHUNK, 16), jnp.float32),
            pltpu.VMEM((CHUNK, 16), jnp.float32),
            pltpu.VMEM((CHUNK, 16), jnp.float32),
            pltpu.VMEM((CHUNK, 16), jnp.float32),
            pltpu.VMEM((CHUNK, 16), jnp.float32),
            pltpu.VMEM((CHUNK, 16), jnp.float32),
            pltpu.VMEM((CHUNK, 16), jnp.float32),
            pltpu.VMEM((N, 16), jnp.float32),
        ] + [pltpu.SemaphoreType.DMA] * 10,
    )(hidden1.reshape(N * 16, 16), rows.reshape(NCHUNKS, CHUNK),
      cols.reshape(NCHUNKS, CHUNK))


def _za_body(fat_ref, w2_ref, wam_ref, was_ref, eps_ref, za_ref):
    h2 = jnp.tanh(
        jax.lax.dot_general(
            fat_ref[...], w2_ref[...], (((1,), (0,)), ((), ())),
            preferred_element_type=jnp.float32,
        )
    )
    mean = jax.lax.dot_general(
        h2, wam_ref[...], (((1,), (0,)), ((), ())),
        preferred_element_type=jnp.float32,
    )
    logstd = jax.lax.dot_general(
        h2, was_ref[...], (((1,), (0,)), ((), ())),
        preferred_element_type=jnp.float32,
    )
    za_ref[...] = mean + eps_ref[...] * jnp.exp(logstd)


def _z_a(FaT, W2, Wam, Was, eps_a):
    return pl.pallas_call(
        _za_body,
        out_shape=jax.ShapeDtypeStruct((F, H2 + L), jnp.float32),
    )(FaT, W2, Wam, Was, eps_a)


BM3 = 512


def _dec_body(p0_ref, p1_ref, prob_ref, lp_ref, yt_ref, gum_ref, eps_ref,
              wum_ref, wus_ref, za_ref, out_ref):
    node_fea = p0_ref[...] + p1_ref[...]
    prob = prob_ref[...]
    lp = lp_ref[...]
    yt = yt_ref[...]
    yz = lp * yt + (1.0 - lp) * prob

    wum = wum_ref[...]
    wus = wus_ref[...]
    z_u_mean = (
        jax.lax.dot_general(node_fea, wum[:H1], (((1,), (0,)), ((), ())),
                            preferred_element_type=jnp.float32)
        + jax.lax.dot_general(yz, wum[H1:], (((1,), (0,)), ((), ())),
                              preferred_element_type=jnp.float32)
    )
    z_u_log = (
        jax.lax.dot_general(node_fea, wus[:H1], (((1,), (0,)), ((), ())),
                            preferred_element_type=jnp.float32)
        + jax.lax.dot_general(yz, wus[H1:], (((1,), (0,)), ((), ())),
                              preferred_element_type=jnp.float32)
    )
    z_u = z_u_mean + eps_ref[...] * jnp.exp(z_u_log)

    g = -jnp.log(-jnp.log(gum_ref[...]))
    yp = jnp.exp((jnp.log(prob) + g) / TEMP)
    yp = yp / jnp.sum(yp, axis=1, keepdims=True)
    y_rec = lp * yt + (1.0 - lp) * yp

    za = za_ref[...]
    out_ref[...] = (
        jax.lax.dot_general(z_u, za[:, :H2], (((1,), (1,)), ((), ())),
                            preferred_element_type=jnp.float32)
        + jax.lax.dot_general(y_rec, za[:, H2:], (((1,), (1,)), ((), ())),
                              preferred_element_type=jnp.float32)
    )


def _decode(p0, p1, prob, lp, y_train, gumbel_u, eps_u, Wum, Wus, z_a):
    return pl.pallas_call(
        _dec_body,
        grid=(N // BM3,),
        in_specs=[
            pl.BlockSpec((BM3, H1), lambda i: (i, 0)),
            pl.BlockSpec((BM3, H1), lambda i: (i, 0)),
            pl.BlockSpec((BM3, L), lambda i: (i, 0)),
            pl.BlockSpec((BM3, L), lambda i: (i, 0)),
            pl.BlockSpec((BM3, L), lambda i: (i, 0)),
            pl.BlockSpec((BM3, L), lambda i: (i, 0)),
            pl.BlockSpec((BM3, H2), lambda i: (i, 0)),
            pl.BlockSpec((H1 + L, H2), lambda i: (0, 0)),
            pl.BlockSpec((H1 + L, H2), lambda i: (0, 0)),
            pl.BlockSpec((F, H2 + L), lambda i: (0, 0)),
        ],
        out_specs=pl.BlockSpec((BM3, F), lambda i: (i, 0)),
        out_shape=jax.ShapeDtypeStruct((N, F), jnp.float32),
    )(p0, p1, prob, lp, y_train, gumbel_u, eps_u, Wum, Wus, z_a)


def kernel(Fn, FaT, edge_index, labels_pos, y_train, gumbel_u, eps_u, eps_a,
           W1, Wy, W2, Wum, Wus, Wam, Was):
    hidden1, prob = _hidden1(Fn, W1, Wy)
    rows = edge_index[0].astype(jnp.int32)
    cols = edge_index[1].astype(jnp.int32)
    partials = _node_fea_partials(hidden1, rows, cols)
    z_a = _z_a(FaT, W2, Wam, Was, eps_a)
    lp = labels_pos.astype(jnp.float32)
    recon = _decode(partials[0], partials[1], prob, lp, y_train, gumbel_u,
                    eps_u, Wum, Wus, z_a)
    return recon.reshape(-1)
